# Initial kernel scaffold; baseline (speedup 1.0000x reference)
import functools

import jax
import jax.numpy as jnp
from jax import lax
from jax.experimental import pallas as pl
from jax.experimental.pallas import tpu as pltpu
from jax.experimental.pallas import tpu_sc as plsc

_VOCAB = 100000
_D = 64
_BATCH = 4096
_HIST = 50
_KEEP = 0.9

_TOTAL = _BATCH * _HIST
_NW = 32
_BPW = _TOTAL // _NW
_CH = 128
_NCH = _BPW // _CH
_LANES = 16

_scale_const = []


def _dropout_scale():
    if not _scale_const:
        keep = _KEEP
        mask = jax.random.bernoulli(
            jax.random.key(42), p=keep, shape=(_BATCH, _HIST, _D))
        scale = jnp.where(mask, 1.0 / keep, 0.0).reshape(_TOTAL, _D)
        _scale_const.append(scale)
    return _scale_const[0]


_mesh = plsc.VectorSubcoreMesh(core_axis_name="c", subcore_axis_name="s")


@functools.partial(
    pl.kernel,
    out_type=jax.ShapeDtypeStruct((_TOTAL, _D), jnp.float32),
    mesh=_mesh,
    scratch_types=[
        pltpu.VMEM((_NCH, _CH), jnp.int32),
        pltpu.VMEM((_CH, _D), jnp.float32),
        pltpu.VMEM((_CH, _D), jnp.float32),
        pltpu.SemaphoreType.DMA,
    ],
)
def _embed_sc(idx_hbm, table_hbm, scale_hbm, out_hbm,
              idx_v, rows_v, scale_v, sem):
    wid = lax.axis_index("s") * 2 + lax.axis_index("c")
    base = wid * _BPW
    pltpu.sync_copy(idx_hbm.at[pl.ds(wid * _NCH, _NCH)], idx_v)

    def chunk_body(c, carry):
        p0 = base + c * _CH
        gather = pltpu.async_copy(table_hbm.at[idx_v.at[c]], rows_v, sem)
        pltpu.sync_copy(scale_hbm.at[pl.ds(p0, _CH)], scale_v)
        gather.wait()

        def mul_body(i, carry2):
            for j in range(_D // _LANES):
                sl = pl.ds(j * _LANES, _LANES)
                rows_v[i, sl] = rows_v[i, sl] * scale_v[i, sl]
            return carry2

        lax.fori_loop(0, _CH, mul_body, 0, unroll=2)
        pltpu.sync_copy(rows_v, out_hbm.at[pl.ds(p0, _CH)])
        return carry

    lax.fori_loop(0, _NCH, chunk_body, 0)


def kernel(inputs, embedding_encoder):
    idx = inputs.reshape(_NW * _NCH, _CH)
    out = _embed_sc(idx, embedding_encoder, _dropout_scale())
    return out.reshape(_BATCH, _HIST, _D)


# speedup vs baseline: 1.1739x; 1.1739x over previous
import functools

import jax
import jax.numpy as jnp
from jax import lax
from jax.experimental import pallas as pl
from jax.experimental.pallas import tpu as pltpu
from jax.experimental.pallas import tpu_sc as plsc

_VOCAB = 100000
_D = 64
_BATCH = 4096
_HIST = 50
_KEEP = 0.9

_TOTAL = _BATCH * _HIST
_NW = 32
_BPW = _TOTAL // _NW
_CH = 128
_NCH = _BPW // _CH
_LANES = 16

_scale_const = []


def _dropout_scale():
    if not _scale_const:
        keep = _KEEP
        mask = jax.random.bernoulli(
            jax.random.key(42), p=keep, shape=(_BATCH, _HIST, _D))
        scale = jnp.where(mask, 1.0 / keep, 0.0).reshape(_TOTAL, _D)
        _scale_const.append(scale)
    return _scale_const[0]


_mesh = plsc.VectorSubcoreMesh(core_axis_name="c", subcore_axis_name="s")


@functools.partial(
    pl.kernel,
    out_type=jax.ShapeDtypeStruct((_TOTAL, _D), jnp.float32),
    mesh=_mesh,
    scratch_types=[
        pltpu.VMEM((_NCH, _CH), jnp.int32),
        pltpu.VMEM((_CH, _D), jnp.float32),
        pltpu.VMEM((_CH, _D), jnp.float32),
        pltpu.SemaphoreType.DMA,
    ],
    compiler_params=pltpu.CompilerParams(use_tc_tiling_on_sc=False),
)
def _embed_sc(idx_hbm, table_hbm, scale_hbm, out_hbm,
              idx_v, rows_v, scale_v, sem):
    wid = lax.axis_index("s") * 2 + lax.axis_index("c")
    base = wid * _BPW
    pltpu.sync_copy(idx_hbm.at[wid], idx_v)

    def chunk_body(c, carry):
        p0 = base + c * _CH
        gather = pltpu.async_copy(table_hbm.at[idx_v.at[c]], rows_v, sem)
        pltpu.sync_copy(scale_hbm.at[pl.ds(p0, _CH)], scale_v)
        gather.wait()

        def mul_body(i, carry2):
            for j in range(_D // _LANES):
                sl = pl.ds(j * _LANES, _LANES)
                rows_v[i, sl] = rows_v[i, sl] * scale_v[i, sl]
            return carry2

        lax.fori_loop(0, _CH, mul_body, 0, unroll=2)
        pltpu.sync_copy(rows_v, out_hbm.at[pl.ds(p0, _CH)])
        return carry

    lax.fori_loop(0, _NCH, chunk_body, 0)


def kernel(inputs, embedding_encoder):
    idx = inputs.reshape(_NW, _NCH, _CH)
    out = _embed_sc(idx, embedding_encoder, _dropout_scale())
    return out.reshape(_BATCH, _HIST, _D)


# speedup vs baseline: 1.4440x; 1.2301x over previous
import functools

import jax
import jax.numpy as jnp
import numpy as np
from jax import lax
from jax.experimental import pallas as pl
from jax.experimental.pallas import tpu as pltpu
from jax.experimental.pallas import tpu_sc as plsc

_VOCAB = 100000
_D = 64
_BATCH = 4096
_HIST = 50
_KEEP = 0.9

_TOTAL = _BATCH * _HIST
_NW = 32
_BPW = _TOTAL // _NW
_CH = 128
_NCH = _BPW // _CH
_HALF = _NCH // 2
_LANES = 16


def _threefry2x32_np(k0, k1, x0, x1):
    rotations = ((13, 15, 26, 6), (17, 29, 16, 24))

    def rotl(v, r):
        return (v << np.uint32(r)) | (v >> np.uint32(32 - r))

    ks = (np.uint32(k0), np.uint32(k1),
          np.uint32(k0) ^ np.uint32(k1) ^ np.uint32(0x1BD11BDA))
    x0 = x0 + ks[0]
    x1 = x1 + ks[1]
    for i in range(5):
        for r in rotations[i % 2]:
            x0 = x0 + x1
            x1 = rotl(x1, r)
            x1 = x1 ^ x0
        x0 = x0 + ks[(i + 1) % 3]
        x1 = x1 + ks[(i + 2) % 3] + np.uint32(i + 1)
    return x0, x1


def _make_scale():
    size = _TOTAL * _D
    counts2 = np.arange(size, dtype=np.uint32)
    counts1 = np.zeros(size, dtype=np.uint32)
    with np.errstate(over="ignore"):
        o0, o1 = _threefry2x32_np(np.uint32(0), np.uint32(42),
                                  counts1, counts2)
    bits = o0 ^ o1
    floats = ((bits >> np.uint32(9)) | np.uint32(0x3F800000)).view(np.float32)
    mask = (floats - np.float32(1.0)) < np.float32(_KEEP)
    return np.where(mask, np.float32(1.0 / _KEEP), np.float32(0.0))


_SCALE = _make_scale()

_mesh = plsc.VectorSubcoreMesh(core_axis_name="c", subcore_axis_name="s")


@functools.partial(
    pl.kernel,
    out_type=jax.ShapeDtypeStruct((_TOTAL * _D,), jnp.float32),
    mesh=_mesh,
    scratch_types=[
        pltpu.VMEM((_NCH, _CH), jnp.int32),
        pltpu.VMEM((2, _CH, _D), jnp.float32),
        pltpu.VMEM((2, _CH * _D), jnp.float32),
        pltpu.VMEM((2, _CH * _D), jnp.float32),
        pltpu.SemaphoreType.DMA,
        pltpu.SemaphoreType.DMA,
        pltpu.SemaphoreType.DMA,
        pltpu.SemaphoreType.DMA,
        pltpu.SemaphoreType.DMA,
        pltpu.SemaphoreType.DMA,
    ],
    compiler_params=pltpu.CompilerParams(use_tc_tiling_on_sc=False),
)
def _embed_sc(idx_hbm, table_hbm, scale_hbm, out_hbm,
              idx_v, rows_v, scale_v, out_v, gs0, gs1, ss0, ss1, ws0, ws1):
    wid = lax.axis_index("s") * 2 + lax.axis_index("c")
    base = wid * _BPW
    gsem = (gs0, gs1)
    ssem = (ss0, ss1)
    wsem = (ws0, ws1)
    pltpu.sync_copy(idx_hbm.at[wid], idx_v)

    def gather_cp(c, b):
        return pltpu.make_async_copy(
            table_hbm.at[idx_v.at[c]], rows_v.at[b], gsem[b])

    def scale_cp(c, b):
        off = (base + c * _CH) * _D
        return pltpu.make_async_copy(
            scale_hbm.at[pl.ds(off, _CH * _D)], scale_v.at[b], ssem[b])

    def write_cp(c, b):
        off = (base + c * _CH) * _D
        return pltpu.make_async_copy(
            out_v.at[b], out_hbm.at[pl.ds(off, _CH * _D)], wsem[b])

    for b in (0, 1):
        gather_cp(b, b).start()
        scale_cp(b, b).start()

    def pair_body(i, carry):
        for b in (0, 1):
            c = 2 * i + b
            gather_cp(c, b).wait()
            scale_cp(c, b).wait()

            @pl.when(i >= 1)
            def _():
                write_cp(c, b).wait()

            def mul_body(r, carry2):
                for j in range(_D // _LANES):
                    o = pl.ds(r * _D + j * _LANES, _LANES)
                    out_v[b, o] = (rows_v[b, r, pl.ds(j * _LANES, _LANES)]
                                   * scale_v[b, o])
                return carry2

            lax.fori_loop(0, _CH, mul_body, 0, unroll=4)
            write_cp(c, b).start()

            @pl.when(i < _HALF - 1)
            def _():
                gather_cp(c + 2, b).start()
                scale_cp(c + 2, b).start()
        return carry

    lax.fori_loop(0, _HALF, pair_body, 0)
    for b in (0, 1):
        write_cp(b, b).wait()


def kernel(inputs, embedding_encoder):
    idx = inputs.reshape(_NW, _NCH, _CH)
    out = _embed_sc(idx, embedding_encoder, _SCALE)
    return out.reshape(_BATCH, _HIST, _D)


# speedup vs baseline: 2.5971x; 1.7986x over previous
import functools

import jax
import jax.numpy as jnp
import numpy as np
from jax import lax
from jax.experimental import pallas as pl
from jax.experimental.pallas import tpu as pltpu
from jax.experimental.pallas import tpu_sc as plsc

_VOCAB = 100000
_D = 64
_BATCH = 4096
_HIST = 50
_KEEP = 0.9

_TOTAL = _BATCH * _HIST
_NW = 32
_BPW = _TOTAL // _NW
_CH = 128
_NCH = _BPW // _CH
_HALF = _NCH // 2
_LANES = 16


def _threefry2x32_np(k0, k1, x0, x1):
    rotations = ((13, 15, 26, 6), (17, 29, 16, 24))

    def rotl(v, r):
        return (v << np.uint32(r)) | (v >> np.uint32(32 - r))

    ks = (np.uint32(k0), np.uint32(k1),
          np.uint32(k0) ^ np.uint32(k1) ^ np.uint32(0x1BD11BDA))
    x0 = x0 + ks[0]
    x1 = x1 + ks[1]
    for i in range(5):
        for r in rotations[i % 2]:
            x0 = x0 + x1
            x1 = rotl(x1, r)
            x1 = x1 ^ x0
        x0 = x0 + ks[(i + 1) % 3]
        x1 = x1 + ks[(i + 2) % 3] + np.uint32(i + 1)
    return x0, x1


def _make_scale():
    size = _TOTAL * _D
    counts2 = np.arange(size, dtype=np.uint32)
    counts1 = np.zeros(size, dtype=np.uint32)
    with np.errstate(over="ignore"):
        o0, o1 = _threefry2x32_np(np.uint32(0), np.uint32(42),
                                  counts1, counts2)
    bits = o0 ^ o1
    floats = ((bits >> np.uint32(9)) | np.uint32(0x3F800000)).view(np.float32)
    mask = (floats - np.float32(1.0)) < np.float32(_KEEP)
    return np.where(mask, np.float32(1.0 / _KEEP),
                    np.float32(0.0)).reshape(_TOTAL, _D)


_SCALE = _make_scale()

_mesh = plsc.VectorSubcoreMesh(core_axis_name="c", subcore_axis_name="s")


@functools.partial(
    pl.kernel,
    out_type=jax.ShapeDtypeStruct((_TOTAL * _D,), jnp.float32),
    mesh=_mesh,
    scratch_types=[
        pltpu.VMEM((_NCH, _CH), jnp.int32),
        pltpu.VMEM((2, _CH, _D), jnp.float32),
        pltpu.VMEM((2, _CH, _D), jnp.float32),
        pltpu.VMEM((2, _CH * _D), jnp.float32),
        pltpu.SemaphoreType.DMA,
        pltpu.SemaphoreType.DMA,
        pltpu.SemaphoreType.DMA,
        pltpu.SemaphoreType.DMA,
        pltpu.SemaphoreType.DMA,
        pltpu.SemaphoreType.DMA,
    ],
    compiler_params=pltpu.CompilerParams(use_tc_tiling_on_sc=False),
)
def _embed_sc(idx_hbm, table_hbm, scale_hbm, out_hbm,
              idx_v, rows_v, scale_v, out_v, gs0, gs1, ss0, ss1, ws0, ws1):
    wid = lax.axis_index("s") * 2 + lax.axis_index("c")
    base = wid * _BPW
    gsem = (gs0, gs1)
    ssem = (ss0, ss1)
    wsem = (ws0, ws1)
    pltpu.sync_copy(idx_hbm.at[wid], idx_v)

    def gather_cp(c, b):
        return pltpu.make_async_copy(
            table_hbm.at[idx_v.at[c]], rows_v.at[b], gsem[b])

    def scale_cp(c, b):
        row0 = base + c * _CH
        return pltpu.make_async_copy(
            scale_hbm.at[pl.ds(row0, _CH)], scale_v.at[b], ssem[b])

    def write_cp(c, b):
        off = (base + c * _CH) * _D
        return pltpu.make_async_copy(
            out_v.at[b], out_hbm.at[pl.ds(off, _CH * _D)], wsem[b])

    for b in (0, 1):
        gather_cp(b, b).start()
        scale_cp(b, b).start()

    def pair_body(i, carry):
        for b in (0, 1):
            c = 2 * i + b
            gather_cp(c, b).wait()
            scale_cp(c, b).wait()

            @pl.when(i >= 1)
            def _():
                write_cp(c, b).wait()

            def mul_body(r, carry2):
                for j in range(_D // _LANES):
                    js = pl.ds(j * _LANES, _LANES)
                    o = pl.ds(r * _D + j * _LANES, _LANES)
                    out_v[b, o] = rows_v[b, r, js] * scale_v[b, r, js]
                return carry2

            lax.fori_loop(0, _CH, mul_body, 0, unroll=4)
            write_cp(c, b).start()

            @pl.when(i < _HALF - 1)
            def _():
                gather_cp(c + 2, b).start()
                scale_cp(c + 2, b).start()
        return carry

    lax.fori_loop(0, _HALF, pair_body, 0)
    for b in (0, 1):
        write_cp(b, b).wait()


def kernel(inputs, embedding_encoder):
    idx = inputs.reshape(_NW, _NCH, _CH)
    out = _embed_sc(idx, embedding_encoder, _SCALE)
    return out.reshape(_BATCH, _HIST, _D)


# speedup vs baseline: 3.6949x; 1.4227x over previous
import functools

import jax
import jax.numpy as jnp
import numpy as np
from jax import lax
from jax.experimental import pallas as pl
from jax.experimental.pallas import tpu as pltpu
from jax.experimental.pallas import tpu_sc as plsc

_VOCAB = 100000
_D = 64
_BATCH = 4096
_HIST = 50
_KEEP = 0.9

_NW = 32
_CH = 128
_NCH = _HIST
_HALF = _NCH // 2
_LANES = 16
_TRS = _D // 8
_PAD = 2 * _LANES * 4 + 1


def _threefry2x32_np(k0, k1, x0, x1):
    rotations = ((13, 15, 26, 6), (17, 29, 16, 24))

    def rotl(v, r):
        return (v << np.uint32(r)) | (v >> np.uint32(32 - r))

    ks = (np.uint32(k0), np.uint32(k1),
          np.uint32(k0) ^ np.uint32(k1) ^ np.uint32(0x1BD11BDA))
    x0 = x0 + ks[0]
    x1 = x1 + ks[1]
    for i in range(5):
        for r in rotations[i % 2]:
            x0 = x0 + x1
            x1 = rotl(x1, r)
            x1 = x1 ^ x0
        x0 = x0 + ks[(i + 1) % 3]
        x1 = x1 + ks[(i + 2) % 3] + np.uint32(i + 1)
    return x0, x1


def _make_scale():
    size = _BATCH * _HIST * _D
    counts2 = np.arange(size, dtype=np.uint32)
    counts1 = np.zeros(size, dtype=np.uint32)
    with np.errstate(over="ignore"):
        o0, o1 = _threefry2x32_np(np.uint32(0), np.uint32(42),
                                  counts1, counts2)
    bits = o0 ^ o1
    floats = ((bits >> np.uint32(9)) | np.uint32(0x3F800000)).view(np.float32)
    mask = (floats - np.float32(1.0)) < np.float32(_KEEP)
    scale = np.where(mask, np.float32(1.0 / _KEEP), np.float32(0.0))
    scale = scale.reshape(_NW, _CH, _HIST, _D).transpose(0, 2, 1, 3)
    return np.ascontiguousarray(scale).reshape(_NW * _HIST, _CH, _D)


_SCALE = _make_scale()

_mesh = plsc.VectorSubcoreMesh(core_axis_name="c", subcore_axis_name="s")


@functools.partial(
    pl.kernel,
    out_type=jax.ShapeDtypeStruct((_HIST, _TRS, _NW, 8, _CH), jnp.float32),
    mesh=_mesh,
    scratch_types=[
        pltpu.VMEM((_NCH, _CH), jnp.int32),
        pltpu.VMEM((2, _CH, _D), jnp.float32),
        pltpu.VMEM((2, _CH, _D), jnp.float32),
        pltpu.VMEM((2, _D, _PAD), jnp.float32),
        pltpu.SemaphoreType.DMA,
        pltpu.SemaphoreType.DMA,
        pltpu.SemaphoreType.DMA,
        pltpu.SemaphoreType.DMA,
        pltpu.SemaphoreType.DMA,
        pltpu.SemaphoreType.DMA,
    ],
    compiler_params=pltpu.CompilerParams(use_tc_tiling_on_sc=False,
                                         needs_layout_passes=False),
)
def _embed_sc(idx_hbm, table_hbm, scale_hbm, out_hbm,
              idx_v, rows_v, scale_v, out_t, gs0, gs1, ss0, ss1, ws0, ws1):
    wid = lax.axis_index("s") * 2 + lax.axis_index("c")
    gsem = (gs0, gs1)
    ssem = (ss0, ss1)
    wsem = (ws0, ws1)
    pltpu.sync_copy(idx_hbm.at[wid], idx_v)

    didx = [lax.iota(jnp.int32, _LANES) + _LANES * j
            for j in range(_D // _LANES)]

    def gather_cp(c, b):
        return pltpu.make_async_copy(
            table_hbm.at[idx_v.at[c]], rows_v.at[b], gsem[b])

    def scale_cp(c, b):
        return pltpu.make_async_copy(
            scale_hbm.at[wid * _HIST + c], scale_v.at[b], ssem[b])

    def write_cps(c, b):
        return [pltpu.make_async_copy(
                    out_t.at[b, pl.ds(8 * tr, 8), pl.ds(0, _CH)],
                    out_hbm.at[c, tr, wid], wsem[b])
                for tr in range(_TRS)]

    for b in (0, 1):
        gather_cp(b, b).start()
        scale_cp(b, b).start()

    def pair_body(i, carry):
        for b in (0, 1):
            c = 2 * i + b
            gather_cp(c, b).wait()
            scale_cp(c, b).wait()

            @pl.when(i >= 1)
            def _():
                for cp in write_cps(c, b):
                    cp.wait()

            def mul_body(r, carry2):
                col = jnp.full((_LANES,), r, jnp.int32)
                for j in range(_D // _LANES):
                    js = pl.ds(j * _LANES, _LANES)
                    v = rows_v[b, r, js] * scale_v[b, r, js]
                    plsc.store_scatter(out_t.at[b], [didx[j], col], v)
                return carry2

            lax.fori_loop(0, _CH, mul_body, 0, unroll=4)
            for cp in write_cps(c, b):
                cp.start()

            @pl.when(i < _HALF - 1)
            def _():
                gather_cp(c + 2, b).start()
                scale_cp(c + 2, b).start()
        return carry

    lax.fori_loop(0, _HALF, pair_body, 0)
    for b in (0, 1):
        for cp in write_cps(b, b):
            cp.wait()


def kernel(inputs, embedding_encoder):
    idx = inputs.reshape(_NW, _CH, _HIST).transpose(0, 2, 1)
    out5 = _embed_sc(idx, embedding_encoder, _SCALE)
    return out5.transpose(2, 4, 0, 1, 3).reshape(_BATCH, _HIST, _D)


# speedup vs baseline: 4.8247x; 1.3058x over previous
import functools

import jax
import jax.numpy as jnp
import numpy as np
from jax import lax
from jax.experimental import pallas as pl
from jax.experimental.pallas import tpu as pltpu
from jax.experimental.pallas import tpu_sc as plsc

_VOCAB = 100000
_D = 64
_BATCH = 4096
_HIST = 50
_KEEP = 0.9

_NW = 32
_CH = 128
_NCH = _HIST
_HALF = _NCH // 2
_LANES = 16
_TRS = _D // 8
_PAD = 2 * _LANES * 4 + 1


def _threefry2x32_np(k0, k1, x0, x1):
    rotations = ((13, 15, 26, 6), (17, 29, 16, 24))

    def rotl(v, r):
        return (v << np.uint32(r)) | (v >> np.uint32(32 - r))

    ks = (np.uint32(k0), np.uint32(k1),
          np.uint32(k0) ^ np.uint32(k1) ^ np.uint32(0x1BD11BDA))
    x0 = x0 + ks[0]
    x1 = x1 + ks[1]
    for i in range(5):
        for r in rotations[i % 2]:
            x0 = x0 + x1
            x1 = rotl(x1, r)
            x1 = x1 ^ x0
        x0 = x0 + ks[(i + 1) % 3]
        x1 = x1 + ks[(i + 2) % 3] + np.uint32(i + 1)
    return x0, x1


def _make_mask_bits():
    size = _BATCH * _HIST * _D
    counts2 = np.arange(size, dtype=np.uint32)
    counts1 = np.zeros(size, dtype=np.uint32)
    with np.errstate(over="ignore"):
        o0, o1 = _threefry2x32_np(np.uint32(0), np.uint32(42),
                                  counts1, counts2)
    bits = o0 ^ o1
    floats = ((bits >> np.uint32(9)) | np.uint32(0x3F800000)).view(np.float32)
    mask = (floats - np.float32(1.0)) < np.float32(_KEEP)
    mask = mask.reshape(_NW, _CH, _HIST, _D).transpose(0, 2, 1, 3)
    m = np.ascontiguousarray(mask).reshape(-1, 32).astype(np.uint32)
    words = (m << np.arange(32, dtype=np.uint32)).sum(
        axis=1, dtype=np.uint32)
    return words.reshape(-1, 8, 128)


_MASKBITS = _make_mask_bits()
_RECIP = np.float32(1.0 / _KEEP)

_mesh = plsc.VectorSubcoreMesh(core_axis_name="c", subcore_axis_name="s")


@functools.partial(
    pl.kernel,
    out_type=jax.ShapeDtypeStruct((_HIST, _TRS, _NW, 8, _CH), jnp.float32),
    mesh=_mesh,
    scratch_types=[
        pltpu.VMEM((_NCH, _CH), jnp.int32),
        pltpu.VMEM((2, _CH, _D), jnp.float32),
        pltpu.VMEM((2, 2, _CH), jnp.uint32),
        pltpu.VMEM((2, _D, _PAD), jnp.float32),
        pltpu.SemaphoreType.DMA,
        pltpu.SemaphoreType.DMA,
        pltpu.SemaphoreType.DMA,
        pltpu.SemaphoreType.DMA,
        pltpu.SemaphoreType.DMA,
        pltpu.SemaphoreType.DMA,
    ],
    compiler_params=pltpu.CompilerParams(use_tc_tiling_on_sc=False,
                                         needs_layout_passes=False),
)
def _embed_sc(idx_hbm, table_hbm, mask_hbm, out_hbm,
              idx_v, rows_v, mask_v, out_t, gs0, gs1, ms0, ms1, ws0, ws1):
    wid = lax.axis_index("s") * 2 + lax.axis_index("c")
    gsem = (gs0, gs1)
    msem = (ms0, ms1)
    wsem = (ws0, ws1)
    pltpu.sync_copy(idx_hbm.at[wid], idx_v)

    didx = [lax.iota(jnp.int32, _LANES) + _LANES * j
            for j in range(_D // _LANES)]
    sh_lo = lax.iota(jnp.uint32, _LANES)
    sh_hi = sh_lo + jnp.uint32(_LANES)
    shs = (sh_lo, sh_hi, sh_lo, sh_hi)

    def gather_cp(c, b):
        return pltpu.make_async_copy(
            table_hbm.at[idx_v.at[c]], rows_v.at[b], gsem[b])

    def mask_cp(c, b):
        cc = wid * _HIST + c
        return pltpu.make_async_copy(
            mask_hbm.at[cc // 4, pl.ds((cc % 4) * 2, 2)],
            mask_v.at[b], msem[b])

    def write_cps(c, b):
        return [pltpu.make_async_copy(
                    out_t.at[b, pl.ds(8 * tr, 8), pl.ds(0, _CH)],
                    out_hbm.at[c, tr, wid], wsem[b])
                for tr in range(_TRS)]

    for b in (0, 1):
        gather_cp(b, b).start()
        mask_cp(b, b).start()

    def pair_body(i, carry):
        for b in (0, 1):
            c = 2 * i + b
            gather_cp(c, b).wait()
            mask_cp(c, b).wait()

            @pl.when(i >= 1)
            def _():
                for cp in write_cps(c, b):
                    cp.wait()

            def mul_body(g, carry2):
                mv = mask_v[b, g // 8, pl.ds((g % 8) * _LANES, _LANES)]
                for r8 in range(8):
                    r = g * 8 + r8
                    col = jnp.full((_LANES,), r, jnp.int32)
                    w0 = lax.broadcast(mv[2 * r8], (_LANES,))
                    w1 = lax.broadcast(mv[2 * r8 + 1], (_LANES,))
                    words = (w0, w0, w1, w1)
                    for j in range(_D // _LANES):
                        js = pl.ds(j * _LANES, _LANES)
                        bit = lax.shift_right_logical(
                            words[j], shs[j]) & jnp.uint32(1)
                        scale = bit.astype(jnp.float32) * _RECIP
                        v = rows_v[b, r, js] * scale
                        plsc.store_scatter(out_t.at[b], [didx[j], col], v)
                return carry2

            lax.fori_loop(0, _CH // 8, mul_body, 0)
            for cp in write_cps(c, b):
                cp.start()

            @pl.when(i < _HALF - 1)
            def _():
                gather_cp(c + 2, b).start()
                mask_cp(c + 2, b).start()
        return carry

    lax.fori_loop(0, _HALF, pair_body, 0)
    for b in (0, 1):
        for cp in write_cps(b, b):
            cp.wait()


def kernel(inputs, embedding_encoder):
    idx = inputs.reshape(_NW, _CH, _HIST).transpose(0, 2, 1)
    out5 = _embed_sc(idx, embedding_encoder, _MASKBITS)
    return out5.transpose(2, 4, 0, 1, 3).reshape(_BATCH, _HIST, _D)


# speedup vs baseline: 5.0412x; 1.0449x over previous
import functools

import jax
import jax.numpy as jnp
import numpy as np
from jax import lax
from jax.experimental import pallas as pl
from jax.experimental.pallas import tpu as pltpu
from jax.experimental.pallas import tpu_sc as plsc

_VOCAB = 100000
_D = 64
_BATCH = 4096
_HIST = 50
_KEEP = 0.9

_NW = 32
_CH = 128
_NCH = _HIST
_HALF = _NCH // 2
_LANES = 16
_TRS = _D // 8
_PAD = 2 * _LANES * 4 + 1


def _threefry2x32_np(k0, k1, x0, x1):
    rotations = ((13, 15, 26, 6), (17, 29, 16, 24))

    def rotl(v, r):
        return (v << np.uint32(r)) | (v >> np.uint32(32 - r))

    ks = (np.uint32(k0), np.uint32(k1),
          np.uint32(k0) ^ np.uint32(k1) ^ np.uint32(0x1BD11BDA))
    x0 = x0 + ks[0]
    x1 = x1 + ks[1]
    for i in range(5):
        for r in rotations[i % 2]:
            x0 = x0 + x1
            x1 = rotl(x1, r)
            x1 = x1 ^ x0
        x0 = x0 + ks[(i + 1) % 3]
        x1 = x1 + ks[(i + 2) % 3] + np.uint32(i + 1)
    return x0, x1


def _make_mask_bits():
    size = _BATCH * _HIST * _D
    counts2 = np.arange(size, dtype=np.uint32)
    counts1 = np.zeros(size, dtype=np.uint32)
    with np.errstate(over="ignore"):
        o0, o1 = _threefry2x32_np(np.uint32(0), np.uint32(42),
                                  counts1, counts2)
    bits = o0 ^ o1
    floats = ((bits >> np.uint32(9)) | np.uint32(0x3F800000)).view(np.float32)
    mask = (floats - np.float32(1.0)) < np.float32(_KEEP)
    mask = mask.reshape(_NW, _CH, _HIST, _D).transpose(0, 2, 1, 3)
    m = np.ascontiguousarray(mask).reshape(-1, 32).astype(np.uint32)
    words = (m << np.arange(32, dtype=np.uint32)).sum(
        axis=1, dtype=np.uint32)
    return words.reshape(-1, 8, 128)


_MASKBITS = _make_mask_bits()
_RECIP = np.float32(1.0 / _KEEP)

_mesh = plsc.VectorSubcoreMesh(core_axis_name="c", subcore_axis_name="s")


@functools.partial(
    pl.kernel,
    out_type=jax.ShapeDtypeStruct((_HIST, _TRS, _NW, 8, _CH), jnp.float32),
    mesh=_mesh,
    scratch_types=[
        pltpu.VMEM((_NCH, _CH), jnp.int32),
        pltpu.VMEM((2, _CH, _D), jnp.float32),
        pltpu.VMEM((2, 2, _CH), jnp.uint32),
        pltpu.VMEM((2, _TRS, 8, _PAD), jnp.float32),
        pltpu.SemaphoreType.DMA,
        pltpu.SemaphoreType.DMA,
        pltpu.SemaphoreType.DMA,
        pltpu.SemaphoreType.DMA,
        pltpu.SemaphoreType.DMA,
        pltpu.SemaphoreType.DMA,
    ],
    compiler_params=pltpu.CompilerParams(use_tc_tiling_on_sc=False,
                                         needs_layout_passes=False),
)
def _embed_sc(idx_hbm, table_hbm, mask_hbm, out_hbm,
              idx_v, rows_v, mask_v, out_t, gs0, gs1, ms0, ms1, ws0, ws1):
    wid = lax.axis_index("s") * 2 + lax.axis_index("c")
    gsem = (gs0, gs1)
    msem = (ms0, ms1)
    wsem = (ws0, ws1)
    pltpu.sync_copy(idx_hbm.at[wid], idx_v)

    didx = [lax.iota(jnp.int32, _LANES) + _LANES * j
            for j in range(_D // _LANES)]
    sh_lo = lax.iota(jnp.uint32, _LANES)
    sh_hi = sh_lo + jnp.uint32(_LANES)
    shs = (sh_lo, sh_hi, sh_lo, sh_hi)

    def gather_cp(c, b):
        return pltpu.make_async_copy(
            table_hbm.at[idx_v.at[c]], rows_v.at[b], gsem[b])

    def mask_cp(c, b):
        cc = wid * _HIST + c
        return pltpu.make_async_copy(
            mask_hbm.at[cc // 4, pl.ds((cc % 4) * 2, 2)],
            mask_v.at[b], msem[b])

    def write_cp(c, b):
        return pltpu.make_async_copy(
            out_t.at[b, pl.ds(0, _TRS), pl.ds(0, 8), pl.ds(0, _CH)],
            out_hbm.at[c, pl.ds(0, _TRS), wid], wsem[b])

    for b in (0, 1):
        gather_cp(b, b).start()
        mask_cp(b, b).start()

    def pair_body(i, carry):
        for b in (0, 1):
            c = 2 * i + b
            gather_cp(c, b).wait()
            mask_cp(c, b).wait()

            @pl.when(i >= 1)
            def _():
                write_cp(c, b).wait()

            def mul_body(g, carry2):
                mv = mask_v[b, g // 8, pl.ds((g % 8) * _LANES, _LANES)]
                for r8 in range(8):
                    r = g * 8 + r8
                    col = jnp.full((_LANES,), r, jnp.int32)
                    w0 = lax.broadcast(mv[2 * r8], (_LANES,))
                    w1 = lax.broadcast(mv[2 * r8 + 1], (_LANES,))
                    words = (w0, w0, w1, w1)
                    for j in range(_D // _LANES):
                        js = pl.ds(j * _LANES, _LANES)
                        bit = lax.shift_right_logical(
                            words[j], shs[j]) & jnp.uint32(1)
                        scale = bit.astype(jnp.float32) * _RECIP
                        v = rows_v[b, r, js] * scale
                        plsc.store_scatter(
                            out_t.at[b], [didx[j] // 8, didx[j] % 8, col], v)
                return carry2

            lax.fori_loop(0, _CH // 8, mul_body, 0)
            write_cp(c, b).start()

            @pl.when(i < _HALF - 1)
            def _():
                gather_cp(c + 2, b).start()
                mask_cp(c + 2, b).start()
        return carry

    lax.fori_loop(0, _HALF, pair_body, 0)
    for b in (0, 1):
        write_cp(b, b).wait()


def kernel(inputs, embedding_encoder):
    idx = inputs.reshape(_NW, _CH, _HIST).transpose(0, 2, 1)
    out5 = _embed_sc(idx, embedding_encoder, _MASKBITS)
    return out5.transpose(2, 4, 0, 1, 3).reshape(_BATCH, _HIST, _D)
